# baseline (speedup 1.0000x reference)
import jax
import jax.numpy as jnp
from jax import lax
from jax.experimental import pallas as pl
from jax.experimental.pallas import tpu as pltpu
from jax.experimental.pallas import tpu_sc as plsc

NX = 24
NY = 24
P = NX * NY
E = 768
B = 64
NC = 2
NS = 16
NW = NC * NS
PPW = P // NW
CH = PPW * E
LANES = 16
NCHUNK = CH // LANES
UNROLL = 8
NBUF = 4


def _sc_kernel(x_hbm, row_hbm, col_hbm, out_hbm,
               row_v, col_v, pos_v,
               buf0, buf1, buf2, buf3,
               isem0, isem1, isem2, isem3,
               osem0, osem1, osem2, osem3):
    bufs = (buf0, buf1, buf2, buf3)
    isems = (isem0, isem1, isem2, isem3)
    osems = (osem0, osem1, osem2, osem3)

    wid = lax.axis_index("s") * NC + lax.axis_index("c")
    base = wid * CH

    def in_copy(b, i):
        src = b * (P * E) + base
        return pltpu.make_async_copy(x_hbm.at[pl.ds(src, CH)], bufs[i], isems[i])

    def out_copy(b, i):
        dst = b * (P * E) + base
        return pltpu.make_async_copy(bufs[i], out_hbm.at[pl.ds(dst, CH)], osems[i])

    for i in range(NBUF):
        in_copy(i, i).start()

    pltpu.sync_copy(row_hbm, row_v)
    pltpu.sync_copy(col_hbm, col_v)

    def pos_row(i, _):
        p = wid * PPW + i
        r = p // NY
        c = p - r * NY

        def pos_chunk(j, _):
            off = i * E + j * LANES
            pos_v[pl.ds(off, LANES)] = (
                row_v[pl.ds(r * E + j * LANES, LANES)]
                + col_v[pl.ds(c * E + j * LANES, LANES)]
            )
            return 0

        lax.fori_loop(0, E // LANES, pos_chunk, 0)
        return 0

    lax.fori_loop(0, PPW, pos_row, 0)

    def round_body(t, _):
        for i in range(NBUF):
            b = NBUF * t + i
            in_copy(b, i).wait()

            @plsc.parallel_loop(0, CH, LANES, unroll=UNROLL)
            def add_body(off):
                plsc.addupdate(bufs[i].at[pl.ds(off, LANES)],
                               pos_v[pl.ds(off, LANES)])

            out_copy(b, i).start()

            j2 = (i + 2) % NBUF
            br = b - 2

            @pl.when(jnp.logical_and(br >= 0, b + 2 < B))
            def _():
                out_copy(br, j2).wait()
                in_copy(b + 2, j2).start()

        return 0

    lax.fori_loop(0, B // NBUF, round_body, 0)

    for i in range(NBUF):
        out_copy(B - NBUF + i, i).wait()


@jax.jit
def _run(xf, rf, cf):
    mesh = plsc.VectorSubcoreMesh(core_axis_name="c", subcore_axis_name="s")
    return pl.kernel(
        _sc_kernel,
        mesh=mesh,
        out_type=jax.ShapeDtypeStruct((B * P * E,), jnp.float32),
        scratch_types=[
            pltpu.VMEM((NX * E,), jnp.float32),
            pltpu.VMEM((NY * E,), jnp.float32),
            pltpu.VMEM((CH,), jnp.float32),
            pltpu.VMEM((CH,), jnp.float32),
            pltpu.VMEM((CH,), jnp.float32),
            pltpu.VMEM((CH,), jnp.float32),
            pltpu.VMEM((CH,), jnp.float32),
            pltpu.SemaphoreType.DMA,
            pltpu.SemaphoreType.DMA,
            pltpu.SemaphoreType.DMA,
            pltpu.SemaphoreType.DMA,
            pltpu.SemaphoreType.DMA,
            pltpu.SemaphoreType.DMA,
            pltpu.SemaphoreType.DMA,
            pltpu.SemaphoreType.DMA,
        ],
    )(xf, rf, cf)


def kernel(x, row_embed, col_embed):
    out = _run(x.reshape(-1), row_embed.reshape(-1), col_embed.reshape(-1))
    return out.reshape(x.shape)


# speedup vs baseline: 2.9700x; 2.9700x over previous
import jax
import jax.numpy as jnp
from jax import lax
from jax.experimental import pallas as pl
from jax.experimental.pallas import tpu as pltpu
from jax.experimental.pallas import tpu_sc as plsc

NX = 24
NY = 24
P = NX * NY
E = 768
B = 64
NBG = 4
NTG = 8
BPG = B // NBG
PPG = P // NTG
PC = 24
NCH = PPG // PC
NIT = BPG * NCH
LANES = 16
UNROLL = 8
NBUF = 4


def _sc_kernel(x_hbm, row_hbm, col_hbm, out_hbm,
               pos_v,
               buf0, buf1, buf2, buf3,
               isem0, isem1, isem2, isem3,
               osem0, osem1, osem2, osem3):
    bufs = (buf0, buf1, buf2, buf3)
    isems = (isem0, isem1, isem2, isem3)
    osems = (osem0, osem1, osem2, osem3)

    wid = lax.axis_index("s") * 2 + lax.axis_index("c")
    gb = wid // NTG
    gt = wid - gb * NTG
    b0 = gb * BPG
    p0 = gt * PPG

    def in_copy(n, i):
        b = b0 + n // NCH
        q = p0 + (n % NCH) * PC
        return pltpu.make_async_copy(
            x_hbm.at[b, pl.ds(q, PC), :], bufs[i], isems[i])

    def out_copy(n, i):
        b = b0 + n // NCH
        q = p0 + (n % NCH) * PC
        return pltpu.make_async_copy(
            bufs[i], out_hbm.at[b, pl.ds(q, PC), :], osems[i])

    pltpu.sync_copy(row_hbm, buf0)
    pltpu.sync_copy(col_hbm, buf1)

    def pos_row(i, _):
        p = p0 + i
        r = p // NY
        c = p - r * NY

        @plsc.parallel_loop(0, E, LANES, unroll=UNROLL)
        def pos_chunk(j):
            pos_v[i, pl.ds(j, LANES)] = (
                buf0[r, pl.ds(j, LANES)] + buf1[c, pl.ds(j, LANES)]
            )

        return 0

    lax.fori_loop(0, PPG, pos_row, 0)

    for i in range(NBUF):
        in_copy(i, i).start()

    def round_body(t, _):
        for i in range(NBUF):
            n = NBUF * t + i
            in_copy(n, i).wait()
            prow = (n % NCH) * PC

            @plsc.parallel_loop(0, PC * E, LANES, unroll=UNROLL)
            def add_body(off):
                r = off // E
                c = off - r * E
                plsc.addupdate(bufs[i].at[r, pl.ds(c, LANES)],
                               pos_v[prow + r, pl.ds(c, LANES)])

            out_copy(n, i).start()

            j2 = (i + 2) % NBUF
            nr = n - 2

            @pl.when(jnp.logical_and(nr >= 0, n + 2 < NIT))
            def _():
                out_copy(nr, j2).wait()
                in_copy(n + 2, j2).start()

        return 0

    lax.fori_loop(0, NIT // NBUF, round_body, 0)

    for i in range(NBUF):
        out_copy(NIT - NBUF + i, i).wait()


@jax.jit
def _run(x, rows, cols):
    mesh = plsc.VectorSubcoreMesh(core_axis_name="c", subcore_axis_name="s")
    return pl.kernel(
        _sc_kernel,
        mesh=mesh,
        out_type=jax.ShapeDtypeStruct((B, P, E), jnp.float32),
        scratch_types=[
            pltpu.VMEM((PPG, E), jnp.float32),
            pltpu.VMEM((PC, E), jnp.float32),
            pltpu.VMEM((PC, E), jnp.float32),
            pltpu.VMEM((PC, E), jnp.float32),
            pltpu.VMEM((PC, E), jnp.float32),
            pltpu.SemaphoreType.DMA,
            pltpu.SemaphoreType.DMA,
            pltpu.SemaphoreType.DMA,
            pltpu.SemaphoreType.DMA,
            pltpu.SemaphoreType.DMA,
            pltpu.SemaphoreType.DMA,
            pltpu.SemaphoreType.DMA,
            pltpu.SemaphoreType.DMA,
        ],
    )(x, rows, cols)


def kernel(x, row_embed, col_embed):
    return _run(x, row_embed, col_embed)


# speedup vs baseline: 3.0396x; 1.0234x over previous
import jax
import jax.numpy as jnp
from jax import lax
from jax.experimental import pallas as pl
from jax.experimental.pallas import tpu as pltpu
from jax.experimental.pallas import tpu_sc as plsc

NX = 24
NY = 24
P = NX * NY
E = 768
B = 64
NBG = 4
NTG = 8
BPG = B // NBG
PPG = P // NTG
PC = 24
NCH = PPG // PC
NIT = BPG * NCH
LANES = 16
UNROLL = 8
NBUF = 4


def _sc_kernel(x_hbm, row_hbm, col_hbm, out_hbm,
               pos_v,
               buf0, buf1, buf2, buf3,
               isem0, isem1, isem2, isem3,
               osem0, osem1, osem2, osem3):
    bufs = (buf0, buf1, buf2, buf3)
    isems = (isem0, isem1, isem2, isem3)
    osems = (osem0, osem1, osem2, osem3)

    wid = lax.axis_index("s") * 2 + lax.axis_index("c")
    gb = wid // NTG
    gt = wid - gb * NTG
    b0 = gb * BPG
    p0 = gt * PPG

    def in_copy(n, i):
        b = b0 + n // NCH
        q = p0 + (n % NCH) * PC
        return pltpu.make_async_copy(
            x_hbm.at[b, pl.ds(q, PC), :], bufs[i], isems[i])

    def out_copy(n, i):
        b = b0 + n // NCH
        q = p0 + (n % NCH) * PC
        return pltpu.make_async_copy(
            bufs[i], out_hbm.at[b, pl.ds(q, PC), :], osems[i])

    in_copy(0, 0).start()
    in_copy(1, 1).start()
    pltpu.sync_copy(row_hbm, buf2)
    pltpu.sync_copy(col_hbm, buf3)

    def pos_row(i, _):
        p = p0 + i
        r = p // NY
        c = p - r * NY

        @plsc.parallel_loop(0, E, LANES, unroll=UNROLL)
        def pos_chunk(j):
            pos_v[i, pl.ds(j, LANES)] = (
                buf2[r, pl.ds(j, LANES)] + buf3[c, pl.ds(j, LANES)]
            )

        return 0

    lax.fori_loop(0, PPG, pos_row, 0)

    in_copy(2, 2).start()
    in_copy(3, 3).start()

    def round_body(t, _):
        for i in range(NBUF):
            n = NBUF * t + i
            in_copy(n, i).wait()
            prow = (n % NCH) * PC

            @plsc.parallel_loop(0, PC * E, LANES, unroll=UNROLL)
            def add_body(off):
                r = off // E
                c = off - r * E
                plsc.addupdate(bufs[i].at[r, pl.ds(c, LANES)],
                               pos_v[prow + r, pl.ds(c, LANES)])

            out_copy(n, i).start()

            j2 = (i + 2) % NBUF
            nr = n - 2

            @pl.when(jnp.logical_and(nr >= 0, n + 2 < NIT))
            def _():
                out_copy(nr, j2).wait()
                in_copy(n + 2, j2).start()

        return 0

    lax.fori_loop(0, NIT // NBUF, round_body, 0)

    for i in range(NBUF):
        out_copy(NIT - NBUF + i, i).wait()


@jax.jit
def _run(x, rows, cols):
    mesh = plsc.VectorSubcoreMesh(core_axis_name="c", subcore_axis_name="s")
    return pl.kernel(
        _sc_kernel,
        mesh=mesh,
        out_type=jax.ShapeDtypeStruct((B, P, E), jnp.float32),
        scratch_types=[
            pltpu.VMEM((PPG, E), jnp.float32),
            pltpu.VMEM((PC, E), jnp.float32),
            pltpu.VMEM((PC, E), jnp.float32),
            pltpu.VMEM((PC, E), jnp.float32),
            pltpu.VMEM((PC, E), jnp.float32),
            pltpu.SemaphoreType.DMA,
            pltpu.SemaphoreType.DMA,
            pltpu.SemaphoreType.DMA,
            pltpu.SemaphoreType.DMA,
            pltpu.SemaphoreType.DMA,
            pltpu.SemaphoreType.DMA,
            pltpu.SemaphoreType.DMA,
            pltpu.SemaphoreType.DMA,
        ],
    )(x, rows, cols)


def kernel(x, row_embed, col_embed):
    return _run(x, row_embed, col_embed)


# speedup vs baseline: 3.1447x; 1.0346x over previous
import functools

import jax
import jax.numpy as jnp
from jax import lax
from jax.experimental import pallas as pl
from jax.experimental.pallas import tpu as pltpu
from jax.experimental.pallas import tpu_sc as plsc

NX = 24
NY = 24
P = NX * NY
E = 768
B = 64
LANES = 16
UNROLL = 8


def _pos_sc_kernel(row_hbm, col_hbm, pos_hbm, row_v, col_v, out_v):
    wid = lax.axis_index("s") * 2 + lax.axis_index("c")

    @pl.when(wid < NX)
    def _():
        pltpu.sync_copy(row_hbm.at[wid, :], row_v)
        pltpu.sync_copy(col_hbm, col_v)

        def one_row(c, _):
            @plsc.parallel_loop(0, E, LANES, unroll=UNROLL)
            def chunk(j):
                out_v[c, pl.ds(j, LANES)] = (
                    row_v[pl.ds(j, LANES)] + col_v[c, pl.ds(j, LANES)]
                )

            return 0

        lax.fori_loop(0, NY, one_row, 0)
        pltpu.sync_copy(out_v, pos_hbm.at[pl.ds(NY * wid, NY), :])


def _add_tc_kernel(x_ref, p_ref, o_ref):
    o_ref[...] = x_ref[...] + p_ref[...][None, :, :]


@jax.jit
def _run(x, rows, cols):
    mesh = plsc.VectorSubcoreMesh(core_axis_name="c", subcore_axis_name="s")
    pos = pl.kernel(
        _pos_sc_kernel,
        mesh=mesh,
        out_type=jax.ShapeDtypeStruct((P, E), jnp.float32),
        scratch_types=[
            pltpu.VMEM((E,), jnp.float32),
            pltpu.VMEM((NY, E), jnp.float32),
            pltpu.VMEM((NY, E), jnp.float32),
        ],
    )(rows, cols)

    return pl.pallas_call(
        _add_tc_kernel,
        grid=(B,),
        in_specs=[
            pl.BlockSpec((1, P, E), lambda b: (b, 0, 0)),
            pl.BlockSpec((P, E), lambda b: (0, 0)),
        ],
        out_specs=pl.BlockSpec((1, P, E), lambda b: (b, 0, 0)),
        out_shape=jax.ShapeDtypeStruct((B, P, E), jnp.float32),
        compiler_params=pltpu.CompilerParams(
            dimension_semantics=("arbitrary",),
        ),
    )(x, pos)


def kernel(x, row_embed, col_embed):
    return _run(x, row_embed, col_embed)
